# baseline (speedup 1.0000x reference)
import jax
import jax.numpy as jnp
from jax.experimental import pallas as pl
from jax.experimental.pallas import tpu as pltpu
from jax.experimental.pallas import tpu_sc as plsc

_W = 128
_PW = 13568
_HPS = 5


def _pack_table(table_t, v, d):
    g = -(-v // _PW)

    def body(xa_ref, xb_ref, o_ref):
        xs = jnp.concatenate([xa_ref[...], xb_ref[...]], axis=0)
        o_ref[...] = jnp.transpose(xs, (1, 0))

    return pl.pallas_call(
        body,
        grid=(g,),
        in_specs=[
            pl.BlockSpec((d, _PW // 2), lambda i: (0, 2 * i)),
            pl.BlockSpec((d, _PW // 2), lambda i: (0, 2 * i + 1)),
        ],
        out_specs=pl.BlockSpec((_PW // 2, 2 * d), lambda i: (i, 0)),
        out_shape=jax.ShapeDtypeStruct((g * _PW // 2, 2 * d), table_t.dtype),
    )(table_t, table_t)


def _sc_gather(table, flat_idx, n, d):
    mesh = plsc.VectorSubcoreMesh(core_axis_name="c", subcore_axis_name="s")

    @pl.kernel(
        out_type=jax.ShapeDtypeStruct((n, d), table.dtype),
        mesh=mesh,
        compiler_params=pltpu.CompilerParams(use_tc_tiling_on_sc=False),
    )
    def gather_kernel(tab_hbm, idx_hbm, out_hbm):
        def body(i_vmem, o_vmem):
            pltpu.sync_copy(tab_hbm.at[i_vmem], o_vmem)

        pltpu.emit_pipeline(
            body,
            grid=(n // _W,),
            in_specs=[pl.BlockSpec((_W,), index_map=lambda i: (i,))],
            out_specs=[pl.BlockSpec((_W, d), index_map=lambda i: (i, 0))],
            core_axis_name=("c", "s"),
            dimension_semantics=(pltpu.PARALLEL,),
        )(idx_hbm, out_hbm)

    return gather_kernel(table, flat_idx).reshape(n * d // 128, 128)


def _transpose_out(rows128, b, h, d, hk, k0, acc):
    hb = b // 2
    rpb = b * d // 128

    def body(x_ref, *rest):
        o_ref = rest[-1]
        xt = jnp.transpose(x_ref[...], (1, 0))
        for j in range(_HPS):
            sl = slice(j * rpb, (j + 1) * rpb)
            o_ref[j * d : j * d + d, :hb] = xt[:d, sl]
            o_ref[j * d : j * d + d, hb:] = xt[d:, sl]

    in_specs = [pl.BlockSpec((_HPS * rpb, 128), lambda i: (i, 0))]
    operands = [rows128]
    aliases = {}
    if acc is not None:
        in_specs.append(pl.BlockSpec(memory_space=pl.ANY))
        operands.append(acc)
        aliases = {1: 0}

    return pl.pallas_call(
        body,
        grid=(hk // _HPS,),
        in_specs=in_specs,
        out_specs=pl.BlockSpec(
            (_HPS * d, b), lambda i, _k0=k0 // _HPS: (i + _k0, 0)
        ),
        out_shape=jax.ShapeDtypeStruct((h * d, b), rows128.dtype),
        input_output_aliases=aliases,
    )(*operands)


def kernel(indices, embeddings):
    b, h = indices.shape
    v, d = embeddings.shape
    n = b * h

    idx_p = (
        indices.astype(jnp.int32)
        .reshape(2, b // 2, h)
        .transpose(1, 0, 2)
        .reshape(b, h)
    )
    flat_idx = idx_p.T.reshape(n)
    u = flat_idx % _PW
    flat_idx = flat_idx + jnp.where(u < _PW // 2, u, u - (_PW - 1))

    table_128 = _pack_table(embeddings.T, v, d)
    table_lin = table_128.reshape(table_128.shape[0] * 2, d)

    chunks = (80, 80, 40)
    out_2d = None
    k0 = 0
    for hk in chunks:
        nk = b * hk
        idx_k = jax.lax.slice(flat_idx, (k0 * b,), (k0 * b + nk,))
        rows128 = _sc_gather(table_lin, idx_k, nk, d)
        out_2d = _transpose_out(rows128, b, h, d, hk, k0, out_2d)
        k0 += hk

    out_t = out_2d.reshape(h, d, b)
    return jnp.transpose(out_t, (2, 0, 1))


# speedup vs baseline: 1.0071x; 1.0071x over previous
import jax
import jax.numpy as jnp
from jax.experimental import pallas as pl
from jax.experimental.pallas import tpu as pltpu
from jax.experimental.pallas import tpu_sc as plsc

_W = 128
_PW = 18688
_HPS = 5


def _pack_table(table_t, v, d):
    g = -(-v // _PW)

    def body(xa_ref, xb_ref, o_ref):
        xs = jnp.concatenate([xa_ref[...], xb_ref[...]], axis=0)
        o_ref[...] = jnp.transpose(xs, (1, 0))

    return pl.pallas_call(
        body,
        grid=(g,),
        in_specs=[
            pl.BlockSpec((d, _PW // 2), lambda i: (0, 2 * i)),
            pl.BlockSpec((d, _PW // 2), lambda i: (0, 2 * i + 1)),
        ],
        out_specs=pl.BlockSpec((_PW // 2, 2 * d), lambda i: (i, 0)),
        out_shape=jax.ShapeDtypeStruct((g * _PW // 2, 2 * d), table_t.dtype),
    )(table_t, table_t)


def _sc_gather(table, flat_idx, n, d):
    mesh = plsc.VectorSubcoreMesh(core_axis_name="c", subcore_axis_name="s")

    @pl.kernel(
        out_type=jax.ShapeDtypeStruct((n, d), table.dtype),
        mesh=mesh,
        compiler_params=pltpu.CompilerParams(use_tc_tiling_on_sc=False),
    )
    def gather_kernel(tab_hbm, idx_hbm, out_hbm):
        def body(i_vmem, o_vmem):
            pltpu.sync_copy(tab_hbm.at[i_vmem], o_vmem)

        pltpu.emit_pipeline(
            body,
            grid=(n // _W,),
            in_specs=[pl.BlockSpec((_W,), index_map=lambda i: (i,))],
            out_specs=[pl.BlockSpec((_W, d), index_map=lambda i: (i, 0))],
            core_axis_name=("c", "s"),
            dimension_semantics=(pltpu.PARALLEL,),
        )(idx_hbm, out_hbm)

    return gather_kernel(table, flat_idx).reshape(n * d // 128, 128)


def _transpose_out(rows128, b, h, d, hk, k0, acc):
    hb = b // 2
    rpb = b * d // 128

    def body(x_ref, *rest):
        o_ref = rest[-1]
        xt = jnp.transpose(x_ref[...], (1, 0))
        for j in range(_HPS):
            sl = slice(j * rpb, (j + 1) * rpb)
            o_ref[j * d : j * d + d, :hb] = xt[:d, sl]
            o_ref[j * d : j * d + d, hb:] = xt[d:, sl]

    in_specs = [pl.BlockSpec((_HPS * rpb, 128), lambda i: (i, 0))]
    operands = [rows128]
    aliases = {}
    if acc is not None:
        in_specs.append(pl.BlockSpec(memory_space=pl.ANY))
        operands.append(acc)
        aliases = {1: 0}

    return pl.pallas_call(
        body,
        grid=(hk // _HPS,),
        in_specs=in_specs,
        out_specs=pl.BlockSpec(
            (_HPS * d, b), lambda i, _k0=k0 // _HPS: (i + _k0, 0)
        ),
        out_shape=jax.ShapeDtypeStruct((h * d, b), rows128.dtype),
        input_output_aliases=aliases,
    )(*operands)


def kernel(indices, embeddings):
    b, h = indices.shape
    v, d = embeddings.shape
    n = b * h

    idx_p = (
        indices.astype(jnp.int32)
        .reshape(2, b // 2, h)
        .transpose(1, 0, 2)
        .reshape(b, h)
    )
    flat_idx = idx_p.T.reshape(n)
    u = flat_idx % _PW
    flat_idx = flat_idx + jnp.where(u < _PW // 2, u, u - (_PW - 1))

    table_128 = _pack_table(embeddings.T, v, d)
    table_lin = table_128.reshape(table_128.shape[0] * 2, d)

    chunks = (130, 70)
    out_2d = None
    k0 = 0
    for hk in chunks:
        nk = b * hk
        idx_k = jax.lax.slice(flat_idx, (k0 * b,), (k0 * b + nk,))
        rows128 = _sc_gather(table_lin, idx_k, nk, d)
        out_2d = _transpose_out(rows128, b, h, d, hk, k0, out_2d)
        k0 += hk

    out_t = out_2d.reshape(h, d, b)
    return jnp.transpose(out_t, (2, 0, 1))


# speedup vs baseline: 1.0148x; 1.0077x over previous
import jax
import jax.numpy as jnp
from jax.experimental import pallas as pl
from jax.experimental.pallas import tpu as pltpu
from jax.experimental.pallas import tpu_sc as plsc

_W = 128
_PW = 18688
_HPS = 5


def _pack_table(table_t, v, d):
    g = -(-v // _PW)

    def body(xa_ref, xb_ref, o_ref):
        xs = jnp.concatenate([xa_ref[...], xb_ref[...]], axis=0)
        o_ref[...] = jnp.transpose(xs, (1, 0))

    return pl.pallas_call(
        body,
        grid=(g,),
        in_specs=[
            pl.BlockSpec((d, _PW // 2), lambda i: (0, 2 * i)),
            pl.BlockSpec((d, _PW // 2), lambda i: (0, 2 * i + 1)),
        ],
        out_specs=pl.BlockSpec((_PW // 2, 2 * d), lambda i: (i, 0)),
        out_shape=jax.ShapeDtypeStruct((g * _PW // 2, 2 * d), table_t.dtype),
    )(table_t, table_t)


def _sc_gather(table, flat_idx, n, d):
    mesh = plsc.VectorSubcoreMesh(core_axis_name="c", subcore_axis_name="s")

    @pl.kernel(
        out_type=jax.ShapeDtypeStruct((n, d), table.dtype),
        mesh=mesh,
        compiler_params=pltpu.CompilerParams(use_tc_tiling_on_sc=False),
    )
    def gather_kernel(tab_hbm, idx_hbm, out_hbm):
        def body(i_vmem, o_vmem):
            pltpu.sync_copy(tab_hbm.at[i_vmem], o_vmem)

        pltpu.emit_pipeline(
            body,
            grid=(n // _W,),
            in_specs=[pl.BlockSpec((_W,), index_map=lambda i: (i,))],
            out_specs=[pl.BlockSpec((_W, d), index_map=lambda i: (i, 0))],
            core_axis_name=("c", "s"),
            dimension_semantics=(pltpu.PARALLEL,),
        )(idx_hbm, out_hbm)

    return gather_kernel(table, flat_idx).reshape(n * d // 128, 128)


def _transpose_out(rows128, b, h, d, hk, k0, acc):
    hb = b // 2
    rpb = b * d // 128

    def body(x_ref, *rest):
        o_ref = rest[-1]
        xt = jnp.transpose(x_ref[...], (1, 0))
        for j in range(_HPS):
            sl = slice(j * rpb, (j + 1) * rpb)
            o_ref[j * d : j * d + d, :hb] = xt[:d, sl]
            o_ref[j * d : j * d + d, hb:] = xt[d:, sl]

    in_specs = [pl.BlockSpec((_HPS * rpb, 128), lambda i: (i, 0))]
    operands = [rows128]
    aliases = {}
    if acc is not None:
        in_specs.append(pl.BlockSpec(memory_space=pl.ANY))
        operands.append(acc)
        aliases = {1: 0}

    return pl.pallas_call(
        body,
        grid=(hk // _HPS,),
        in_specs=in_specs,
        out_specs=pl.BlockSpec(
            (_HPS * d, b), lambda i, _k0=k0 // _HPS: (i + _k0, 0)
        ),
        out_shape=jax.ShapeDtypeStruct((h * d, b), rows128.dtype),
        input_output_aliases=aliases,
    )(*operands)


def kernel(indices, embeddings):
    b, h = indices.shape
    v, d = embeddings.shape
    n = b * h

    idx_p = (
        indices.astype(jnp.int32)
        .reshape(2, b // 2, h)
        .transpose(1, 0, 2)
        .reshape(b, h)
    )
    flat_idx = idx_p.T.reshape(n)
    u = flat_idx % _PW
    flat_idx = flat_idx + jnp.where(u < _PW // 2, u, u - (_PW - 1))

    table_128 = _pack_table(embeddings.T, v, d)
    table_lin = table_128.reshape(table_128.shape[0] * 2, d)

    chunks = (90, 80, 30)
    out_2d = None
    k0 = 0
    for hk in chunks:
        nk = b * hk
        idx_k = jax.lax.slice(flat_idx, (k0 * b,), (k0 * b + nk,))
        rows128 = _sc_gather(table_lin, idx_k, nk, d)
        out_2d = _transpose_out(rows128, b, h, d, hk, k0, out_2d)
        k0 += hk

    out_t = out_2d.reshape(h, d, b)
    return jnp.transpose(out_t, (2, 0, 1))
